# baseline (speedup 1.0000x reference)
import functools

import jax
import jax.numpy as jnp
from jax import lax
from jax.experimental import pallas as pl
from jax.experimental.pallas import tpu as pltpu
from jax.experimental.pallas import tpu_sc as plsc


def _qq_xla_order(q):
    return (q[:, 0:1] * q[:, 0:1] + q[:, 2:3] * q[:, 2:3]) + q[:, 1:2] * q[:, 1:2]


def _knn_kernel(q_ref, st_ref, idx_ref, dv_ref, d_ref, *, k, nsrc):
    q = q_ref[...]
    st = st_ref[...]
    qq = _qq_xla_order(q)
    ss = jnp.sum(st * st, axis=0, keepdims=True)
    dot = jnp.dot(q, st, preferred_element_type=jnp.float32)
    d_ref[...] = qq + ss - 2.0 * dot

    bq = q.shape[0]
    kiota = lax.broadcasted_iota(jnp.int32, (bq, k), 1)

    def body(j, carry):
        idxs, dvs = carry
        d = d_ref[...]
        iota = lax.broadcasted_iota(jnp.int32, (bq, nsrc), 1)
        rowmin = jnp.min(d, axis=1, keepdims=True)
        am = jnp.min(jnp.where(d == rowmin, iota, nsrc),
                     axis=1, keepdims=True)
        d_ref[...] = jnp.where(iota == am, 1e30, d)
        sel = kiota == j
        idxs = jnp.where(sel, am, idxs)
        dvs = jnp.where(sel, rowmin, dvs)
        return idxs, dvs

    idxs0 = jnp.zeros((bq, k), jnp.int32)
    dvs0 = jnp.zeros((bq, k), jnp.float32)
    idxs, dvs = lax.fori_loop(0, k, body, (idxs0, dvs0))
    idx_ref[...] = idxs
    dv_ref[...] = jnp.maximum(dvs, 0.0)


def _knn(q, s, k, bq):
    m, nsrc = q.shape[0], s.shape[0]
    bq = min(bq, m)
    st = s.T
    idx, dv = pl.pallas_call(
        functools.partial(_knn_kernel, k=k, nsrc=nsrc),
        grid=(m // bq,),
        in_specs=[pl.BlockSpec((bq, 3), lambda i: (i, 0)),
                  pl.BlockSpec((3, nsrc), lambda i: (0, 0))],
        out_specs=[pl.BlockSpec((bq, k), lambda i: (i, 0)),
                   pl.BlockSpec((bq, k), lambda i: (i, 0))],
        out_shape=[jax.ShapeDtypeStruct((m, k), jnp.int32),
                   jax.ShapeDtypeStruct((m, k), jnp.float32)],
        scratch_shapes=[pltpu.VMEM((bq, nsrc), jnp.float32)],
    )(q, st)
    return idx, dv


def _knn_fa_kernel(q_ref, st_ref, d_out_ref, selc_ref, gidx_ref, m_ref,
                   *, k, nsrc):
    q = q_ref[...]
    st = st_ref[...]
    qq = _qq_xla_order(q)
    ss = jnp.sum(st * st, axis=0, keepdims=True)
    dot = jnp.dot(q, st, preferred_element_type=jnp.float32)
    d = qq + ss - 2.0 * dot

    nc = nsrc // 128
    bq = q.shape[0]
    d_out_ref[...] = d.reshape(bq * nc, 128)
    for c in range(nc):
        m_ref[:, c:c + 1] = jnp.min(d[:, c * 128:(c + 1) * 128],
                                    axis=1, keepdims=True)
    m = m_ref[...]

    kiota = lax.broadcasted_iota(jnp.int32, (bq, k), 1)
    ciota = lax.broadcasted_iota(jnp.int32, (bq, nc), 1)
    selc = jnp.zeros((bq, k), jnp.int32)
    for j in range(k):
        rowmin = jnp.min(m, axis=1, keepdims=True)
        am = jnp.min(jnp.where(m == rowmin, ciota, nc), axis=1, keepdims=True)
        m = jnp.where(ciota == am, 1e30, m)
        selc = jnp.where(kiota == j, am, selc)
    selc_ref[...] = selc
    qglob = pl.program_id(0) * bq + lax.broadcasted_iota(jnp.int32, (bq, k), 0)
    gidx_ref[...] = qglob * nc + selc


def _knn_fc_kernel(e_ref, selc_ref, idx_ref, dv_ref, *, k, ncand):
    d = e_ref[...]
    selc = selc_ref[...]
    bq = selc.shape[0]
    liota = lax.broadcasted_iota(jnp.int32, (1, 128), 1)
    o = jnp.concatenate(
        [selc[:, a:a + 1] * 128 + liota for a in range(k)], axis=1)

    kiota = lax.broadcasted_iota(jnp.int32, (bq, k), 1)
    idxs = jnp.zeros((bq, k), jnp.int32)
    dvs = jnp.zeros((bq, k), jnp.float32)
    big = jnp.int32(2 ** 30)
    for j in range(k):
        rowmin = jnp.min(d, axis=1, keepdims=True)
        am = jnp.min(jnp.where(d == rowmin, o, big), axis=1, keepdims=True)
        d = jnp.where(o == am, 1e30, d)
        sel = kiota == j
        idxs = jnp.where(sel, am, idxs)
        dvs = jnp.where(sel, rowmin, dvs)
    idx_ref[...] = idxs
    dv_ref[...] = jnp.maximum(dvs, 0.0)


def _knn_filtered(q, s, k, bq, bq2):
    m, nsrc = q.shape[0], s.shape[0]
    nc = nsrc // 128
    ncand = k * 128
    st = s.T
    dmat, selc, gidx = pl.pallas_call(
        functools.partial(_knn_fa_kernel, k=k, nsrc=nsrc),
        grid=(m // bq,),
        in_specs=[pl.BlockSpec((bq, 3), lambda i: (i, 0)),
                  pl.BlockSpec((3, nsrc), lambda i: (0, 0))],
        out_specs=[pl.BlockSpec((bq * nc, 128), lambda i: (i, 0)),
                   pl.BlockSpec((bq, k), lambda i: (i, 0)),
                   pl.BlockSpec((bq, k), lambda i: (i, 0))],
        out_shape=[jax.ShapeDtypeStruct((m * nc, 128), jnp.float32),
                   jax.ShapeDtypeStruct((m, k), jnp.int32),
                   jax.ShapeDtypeStruct((m, k), jnp.int32)],
        scratch_shapes=[pltpu.VMEM((bq, nc), jnp.float32)],
    )(q, st)
    e = _sc_gather(dmat, gidx.reshape(-1))
    ewide = e.reshape(m, ncand)
    return pl.pallas_call(
        functools.partial(_knn_fc_kernel, k=k, ncand=ncand),
        grid=(m // bq2,),
        in_specs=[pl.BlockSpec((bq2, ncand), lambda i: (i, 0)),
                  pl.BlockSpec((bq2, k), lambda i: (i, 0))],
        out_specs=[pl.BlockSpec((bq2, k), lambda i: (i, 0)),
                   pl.BlockSpec((bq2, k), lambda i: (i, 0))],
        out_shape=[jax.ShapeDtypeStruct((m, k), jnp.int32),
                   jax.ShapeDtypeStruct((m, k), jnp.float32)],
    )(ewide, selc)


def _linear_kernel(x_ref, w_ref, b_ref, o_ref):
    o_ref[...] = jnp.dot(x_ref[...], w_ref[...],
                         preferred_element_type=jnp.float32) + b_ref[...]


def _linear(x, w, b, br=512):
    r, cin = x.shape
    cout = w.shape[1]
    br = min(br, r)
    return pl.pallas_call(
        _linear_kernel,
        grid=(r // br,),
        in_specs=[pl.BlockSpec((br, cin), lambda i: (i, 0)),
                  pl.BlockSpec((cin, cout), lambda i: (0, 0)),
                  pl.BlockSpec((1, cout), lambda i: (0, 0))],
        out_specs=pl.BlockSpec((br, cout), lambda i: (i, 0)),
        out_shape=jax.ShapeDtypeStruct((r, cout), jnp.float32),
    )(x, w, b.reshape(1, -1))


def _sc_gather(table, idx):
    v, dd = table.shape
    b = idx.shape[0]
    info = plsc.get_sparse_core_info()
    nw = info.num_cores * info.num_subcores
    b_per_w = b // nw
    chunk = min(b_per_w, 128)
    while chunk > 8 and 2 * chunk * dd + b_per_w > 57344:
        chunk //= 2
    n_chunks = b_per_w // chunk
    mesh = plsc.VectorSubcoreMesh(core_axis_name="c", subcore_axis_name="s")

    @functools.partial(
        pl.kernel, mesh=mesh,
        out_type=jax.ShapeDtypeStruct((b, dd), jnp.float32),
        scratch_types=[
            pltpu.VMEM((b_per_w,), jnp.int32),
            pltpu.VMEM((chunk, dd), jnp.float32),
            pltpu.VMEM((chunk, dd), jnp.float32),
            pltpu.SemaphoreType.DMA,
            pltpu.SemaphoreType.DMA,
        ],
    )
    def gk(table_hbm, idx_hbm, out_hbm, idx_v, rows0, rows1, sem0, sem1):
        wid = lax.axis_index("s") * info.num_cores + lax.axis_index("c")
        base = wid * b_per_w
        pltpu.sync_copy(idx_hbm.at[pl.ds(base, b_per_w)], idx_v)

        def pair(p, carry):
            o0 = 2 * p * chunk
            o1 = o0 + chunk
            g0 = pltpu.async_copy(
                table_hbm.at[idx_v.at[pl.ds(o0, chunk)]], rows0, sem0)
            g1 = pltpu.async_copy(
                table_hbm.at[idx_v.at[pl.ds(o1, chunk)]], rows1, sem1)
            g0.wait()
            pltpu.sync_copy(rows0, out_hbm.at[pl.ds(base + o0, chunk)])
            g1.wait()
            pltpu.sync_copy(rows1, out_hbm.at[pl.ds(base + o1, chunk)])
            return carry

        if n_chunks > 1:
            lax.fori_loop(0, n_chunks // 2, pair, 0)
        if n_chunks % 2:
            o = (n_chunks - 1) * chunk
            pltpu.async_copy(
                table_hbm.at[idx_v.at[pl.ds(o, chunk)]], rows0, sem0).wait()
            pltpu.sync_copy(rows0, out_hbm.at[pl.ds(base + o, chunk)])

    return gk(table, idx)


def _sa_tail_kernel(g_ref, q_ref, wq_ref, w2_ref, b2_ref, w3_ref, b3_ref,
                    o_ref, *, ns):
    qproj = jnp.dot(q_ref[...], wq_ref[...],
                    preferred_element_type=jnp.float32)
    bm, c1 = qproj.shape
    g = g_ref[...]
    x = g.reshape(bm, ns, c1) - qproj[:, None, :]
    x = jnp.maximum(x, 0.0).reshape(bm * ns, c1)
    h = jnp.maximum(jnp.dot(x, w2_ref[...],
                            preferred_element_type=jnp.float32) + b2_ref[...], 0.0)
    h = jnp.maximum(jnp.dot(h, w3_ref[...],
                            preferred_element_type=jnp.float32) + b3_ref[...], 0.0)
    o_ref[...] = jnp.max(h.reshape(bm, ns, h.shape[1]), axis=1)


def _sa_tail(g, new_pos, ws, bs, ns, bm):
    m = new_pos.shape[0]
    bm = min(bm, m)
    c1 = ws[0].shape[1]
    c3 = ws[2].shape[1]
    return pl.pallas_call(
        functools.partial(_sa_tail_kernel, ns=ns),
        grid=(m // bm,),
        in_specs=[pl.BlockSpec((bm * ns, c1), lambda i: (i, 0)),
                  pl.BlockSpec((bm, 3), lambda i: (i, 0)),
                  pl.BlockSpec((3, c1), lambda i: (0, 0)),
                  pl.BlockSpec(ws[1].shape, lambda i: (0, 0)),
                  pl.BlockSpec((1, ws[1].shape[1]), lambda i: (0, 0)),
                  pl.BlockSpec(ws[2].shape, lambda i: (0, 0)),
                  pl.BlockSpec((1, c3), lambda i: (0, 0))],
        out_specs=pl.BlockSpec((bm, c3), lambda i: (i, 0)),
        out_shape=jax.ShapeDtypeStruct((m, c3), jnp.float32),
    )(g, new_pos, ws[0][:3], ws[1], bs[1].reshape(1, -1),
      ws[2], bs[2].reshape(1, -1))


def _sa(pos, feat, idx, ws, bs, bm):
    m = pos.shape[0] // 4
    new_pos = pos[:m]
    w1, b1, w2 = ws[0], bs[0], ws[1]
    c1 = w1.shape[1]
    if c1 % 128:
        pad = 128 - c1 % 128
        w1 = jnp.pad(w1, ((0, 0), (0, pad)))
        b1 = jnp.pad(b1, ((0, pad),))
        w2 = jnp.pad(w2, ((0, pad), (0, 0)))
    h = _linear(jnp.concatenate([pos, feat], axis=1), w1, b1)
    g = _sc_gather(h, idx.reshape(-1))
    f = _sa_tail(g, new_pos, [w1, w2, ws[2]], [b1, bs[1], bs[2]], 32, bm)
    return new_pos, f


def _fp_tail_kernel(*refs, n_layers, has_f1):
    rows_ref, dv_ref = refs[0], refs[1]
    pos_arg = 2
    f1 = None
    if has_f1:
        f1 = refs[pos_arg][...]
        pos_arg += 1
    wb = refs[pos_arg:pos_arg + 2 * n_layers]
    o_ref = refs[-1]

    dv = dv_ref[...]
    recip = 1.0 / (jnp.sqrt(dv) + 1e-8)
    w = recip / jnp.sum(recip, axis=1, keepdims=True)
    rows = rows_ref[...]
    bm = dv.shape[0]
    c2 = rows.shape[1]
    interp = jnp.sum(rows.reshape(bm, 3, c2) * w[:, :, None], axis=1)
    x = interp if f1 is None else jnp.concatenate([f1, interp], axis=1)
    for i in range(n_layers):
        x = jnp.maximum(
            jnp.dot(x, wb[2 * i][...],
                    preferred_element_type=jnp.float32) + wb[2 * i + 1][...],
            0.0)
    o_ref[...] = x


def _fp(pos1, feat1, feat2, idx, dv, ws, bs, bm):
    m = pos1.shape[0]
    rows = _sc_gather(feat2, idx.reshape(-1))
    bm = min(bm, m)
    c2 = feat2.shape[1]
    n_layers = len(ws)
    has_f1 = feat1 is not None
    cout = ws[-1].shape[1]

    in_specs = [pl.BlockSpec((bm * 3, c2), lambda i: (i, 0)),
                pl.BlockSpec((bm, 3), lambda i: (i, 0))]
    args = [rows, dv]
    if has_f1:
        in_specs.append(pl.BlockSpec((bm, feat1.shape[1]), lambda i: (i, 0)))
        args.append(feat1)
    for w, b in zip(ws, bs):
        in_specs.append(pl.BlockSpec(w.shape, lambda i: (0, 0)))
        in_specs.append(pl.BlockSpec((1, w.shape[1]), lambda i: (0, 0)))
        args.append(w)
        args.append(b.reshape(1, -1))

    return pl.pallas_call(
        functools.partial(_fp_tail_kernel, n_layers=n_layers, has_f1=has_f1),
        grid=(m // bm,),
        in_specs=in_specs,
        out_specs=pl.BlockSpec((bm, cout), lambda i: (i, 0)),
        out_shape=jax.ShapeDtypeStruct((m, cout), jnp.float32),
    )(*args)


def kernel(points, batch_size, sa1_w, sa1_b, sa2_w, sa2_b, sa3_w, sa3_b,
           sa4_w, sa4_b, fp4_w, fp4_b, fp3_w, fp3_b, fp2_w, fp2_b,
           fp1_w, fp1_b):
    pos = points[:, 1:4]
    feat = points[:, 4:]
    p1, p2, p3, p4 = pos[:4096], pos[:1024], pos[:256], pos[:64]

    i_sa1, _ = _knn_filtered(p1, pos, 32, 128, 256)
    i_fp1, d_fp1 = _knn_filtered(pos, p1, 3, 256, 1024)
    i_sa2, _ = _knn(p2, p1, 32, 128)
    i_sa3, _ = _knn(p3, p2, 32, 256)
    i_sa4, _ = _knn(p4, p3, 32, 64)
    i_fp2, d_fp2 = _knn(p1, p2, 3, 256)
    i_fp3, d_fp3 = _knn(p2, p3, 3, 256)
    i_fp4, d_fp4 = _knn(p3, p4, 3, 256)

    _, feat1 = _sa(pos, feat, i_sa1, sa1_w, sa1_b, bm=256)
    _, feat2 = _sa(p1, feat1, i_sa2, sa2_w, sa2_b, bm=64)
    _, feat3 = _sa(p2, feat2, i_sa3, sa3_w, sa3_b, bm=64)
    _, feat4 = _sa(p3, feat3, i_sa4, sa4_w, sa4_b, bm=16)
    feat3 = _fp(p3, feat3, feat4, i_fp4, d_fp4, fp4_w, fp4_b, bm=64)
    feat2 = _fp(p2, feat2, feat3, i_fp3, d_fp3, fp3_w, fp3_b, bm=128)
    feat1 = _fp(p1, feat1, feat2, i_fp2, d_fp2, fp2_w, fp2_b, bm=128)
    feat0 = _fp(pos, None, feat1, i_fp1, d_fp1, fp1_w, fp1_b, bm=256)
    return feat0


# speedup vs baseline: 1.0497x; 1.0497x over previous
import functools

import jax
import jax.numpy as jnp
from jax import lax
from jax.experimental import pallas as pl
from jax.experimental.pallas import tpu as pltpu
from jax.experimental.pallas import tpu_sc as plsc


def _qq_xla_order(q):
    return (q[:, 0:1] * q[:, 0:1] + q[:, 2:3] * q[:, 2:3]) + q[:, 1:2] * q[:, 1:2]


def _knn_kernel(q_ref, st_ref, idx_ref, dv_ref, d_ref, *, k, nsrc):
    q = q_ref[...]
    st = st_ref[...]
    qq = _qq_xla_order(q)
    ss = jnp.sum(st * st, axis=0, keepdims=True)
    dot = jnp.dot(q, st, preferred_element_type=jnp.float32)
    d_ref[...] = qq + ss - 2.0 * dot

    bq = q.shape[0]
    kiota = lax.broadcasted_iota(jnp.int32, (bq, k), 1)

    def body(j, carry):
        idxs, dvs = carry
        d = d_ref[...]
        iota = lax.broadcasted_iota(jnp.int32, (bq, nsrc), 1)
        rowmin = jnp.min(d, axis=1, keepdims=True)
        am = jnp.min(jnp.where(d == rowmin, iota, nsrc),
                     axis=1, keepdims=True)
        d_ref[...] = jnp.where(iota == am, 1e30, d)
        sel = kiota == j
        idxs = jnp.where(sel, am, idxs)
        dvs = jnp.where(sel, rowmin, dvs)
        return idxs, dvs

    idxs0 = jnp.zeros((bq, k), jnp.int32)
    dvs0 = jnp.zeros((bq, k), jnp.float32)
    idxs, dvs = lax.fori_loop(0, k, body, (idxs0, dvs0))
    idx_ref[...] = idxs
    dv_ref[...] = jnp.maximum(dvs, 0.0)


def _knn(q, s, k, bq):
    m, nsrc = q.shape[0], s.shape[0]
    bq = min(bq, m)
    st = s.T
    idx, dv = pl.pallas_call(
        functools.partial(_knn_kernel, k=k, nsrc=nsrc),
        grid=(m // bq,),
        in_specs=[pl.BlockSpec((bq, 3), lambda i: (i, 0)),
                  pl.BlockSpec((3, nsrc), lambda i: (0, 0))],
        out_specs=[pl.BlockSpec((bq, k), lambda i: (i, 0)),
                   pl.BlockSpec((bq, k), lambda i: (i, 0))],
        out_shape=[jax.ShapeDtypeStruct((m, k), jnp.int32),
                   jax.ShapeDtypeStruct((m, k), jnp.float32)],
        scratch_shapes=[pltpu.VMEM((bq, nsrc), jnp.float32)],
    )(q, st)
    return idx, dv


def _knn_fa_kernel(q_ref, st_ref, d_out_ref, selc_ref, gidx_ref, m_ref,
                   *, k, nsrc):
    q = q_ref[...]
    st = st_ref[...]
    qq = _qq_xla_order(q)
    ss = jnp.sum(st * st, axis=0, keepdims=True)
    dot = jnp.dot(q, st, preferred_element_type=jnp.float32)
    d = qq + ss - 2.0 * dot

    nc = nsrc // 128
    bq = q.shape[0]
    d_out_ref[...] = d.reshape(bq * nc, 128)
    for c in range(nc):
        m_ref[:, c:c + 1] = jnp.min(d[:, c * 128:(c + 1) * 128],
                                    axis=1, keepdims=True)
    m = m_ref[...]

    kiota = lax.broadcasted_iota(jnp.int32, (bq, k), 1)
    ciota = lax.broadcasted_iota(jnp.int32, (bq, nc), 1)
    selc = jnp.zeros((bq, k), jnp.int32)
    for j in range(k):
        rowmin = jnp.min(m, axis=1, keepdims=True)
        am = jnp.min(jnp.where(m == rowmin, ciota, nc), axis=1, keepdims=True)
        m = jnp.where(ciota == am, 1e30, m)
        selc = jnp.where(kiota == j, am, selc)
    selc_ref[...] = selc
    qglob = pl.program_id(0) * bq + lax.broadcasted_iota(jnp.int32, (bq, k), 0)
    gidx_ref[...] = qglob * nc + selc


def _knn_fc_kernel(e_ref, selc_ref, idx_ref, dv_ref, *, k, ncand):
    d = e_ref[...]
    selc = selc_ref[...]
    bq = selc.shape[0]
    liota = lax.broadcasted_iota(jnp.int32, (1, 128), 1)
    o = jnp.concatenate(
        [selc[:, a:a + 1] * 128 + liota for a in range(k)], axis=1)

    kiota = lax.broadcasted_iota(jnp.int32, (bq, k), 1)
    idxs = jnp.zeros((bq, k), jnp.int32)
    dvs = jnp.zeros((bq, k), jnp.float32)
    big = jnp.int32(2 ** 30)
    for j in range(k):
        rowmin = jnp.min(d, axis=1, keepdims=True)
        am = jnp.min(jnp.where(d == rowmin, o, big), axis=1, keepdims=True)
        d = jnp.where(o == am, 1e30, d)
        sel = kiota == j
        idxs = jnp.where(sel, am, idxs)
        dvs = jnp.where(sel, rowmin, dvs)
    idx_ref[...] = idxs
    dv_ref[...] = jnp.maximum(dvs, 0.0)


def _knn_filtered(q, s, k, bq, bq2):
    m, nsrc = q.shape[0], s.shape[0]
    nc = nsrc // 128
    ncand = k * 128
    st = s.T
    dmat, selc, gidx = pl.pallas_call(
        functools.partial(_knn_fa_kernel, k=k, nsrc=nsrc),
        grid=(m // bq,),
        in_specs=[pl.BlockSpec((bq, 3), lambda i: (i, 0)),
                  pl.BlockSpec((3, nsrc), lambda i: (0, 0))],
        out_specs=[pl.BlockSpec((bq * nc, 128), lambda i: (i, 0)),
                   pl.BlockSpec((bq, k), lambda i: (i, 0)),
                   pl.BlockSpec((bq, k), lambda i: (i, 0))],
        out_shape=[jax.ShapeDtypeStruct((m * nc, 128), jnp.float32),
                   jax.ShapeDtypeStruct((m, k), jnp.int32),
                   jax.ShapeDtypeStruct((m, k), jnp.int32)],
        scratch_shapes=[pltpu.VMEM((bq, nc), jnp.float32)],
    )(q, st)
    e = _sc_gather(dmat, gidx.reshape(-1))
    ewide = e.reshape(m, ncand)
    return pl.pallas_call(
        functools.partial(_knn_fc_kernel, k=k, ncand=ncand),
        grid=(m // bq2,),
        in_specs=[pl.BlockSpec((bq2, ncand), lambda i: (i, 0)),
                  pl.BlockSpec((bq2, k), lambda i: (i, 0))],
        out_specs=[pl.BlockSpec((bq2, k), lambda i: (i, 0)),
                   pl.BlockSpec((bq2, k), lambda i: (i, 0))],
        out_shape=[jax.ShapeDtypeStruct((m, k), jnp.int32),
                   jax.ShapeDtypeStruct((m, k), jnp.float32)],
    )(ewide, selc)


def _linear_kernel(x_ref, w_ref, b_ref, o_ref):
    o_ref[...] = jnp.dot(x_ref[...], w_ref[...],
                         preferred_element_type=jnp.float32) + b_ref[...]


def _linear(x, w, b, br=512):
    r, cin = x.shape
    cout = w.shape[1]
    br = min(br, r)
    return pl.pallas_call(
        _linear_kernel,
        grid=(r // br,),
        in_specs=[pl.BlockSpec((br, cin), lambda i: (i, 0)),
                  pl.BlockSpec((cin, cout), lambda i: (0, 0)),
                  pl.BlockSpec((1, cout), lambda i: (0, 0))],
        out_specs=pl.BlockSpec((br, cout), lambda i: (i, 0)),
        out_shape=jax.ShapeDtypeStruct((r, cout), jnp.float32),
    )(x, w, b.reshape(1, -1))


def _sc_gather(table, idx):
    v, dd = table.shape
    b = idx.shape[0]
    info = plsc.get_sparse_core_info()
    nw = info.num_cores * info.num_subcores
    b_per_w = b // nw
    chunk = min(b_per_w, 128)
    while chunk > 8 and 2 * chunk * dd + b_per_w > 57344:
        chunk //= 2
    n_chunks = b_per_w // chunk
    mesh = plsc.VectorSubcoreMesh(core_axis_name="c", subcore_axis_name="s")

    @functools.partial(
        pl.kernel, mesh=mesh,
        out_type=jax.ShapeDtypeStruct((b, dd), jnp.float32),
        scratch_types=[
            pltpu.VMEM((b_per_w,), jnp.int32),
            pltpu.VMEM((chunk, dd), jnp.float32),
            pltpu.VMEM((chunk, dd), jnp.float32),
            pltpu.SemaphoreType.DMA,
            pltpu.SemaphoreType.DMA,
        ],
    )
    def gk(table_hbm, idx_hbm, out_hbm, idx_v, rows0, rows1, sem0, sem1):
        wid = lax.axis_index("s") * info.num_cores + lax.axis_index("c")
        base = wid * b_per_w
        pltpu.sync_copy(idx_hbm.at[pl.ds(base, b_per_w)], idx_v)

        def pair(p, carry):
            o0 = 2 * p * chunk
            o1 = o0 + chunk
            g0 = pltpu.async_copy(
                table_hbm.at[idx_v.at[pl.ds(o0, chunk)]], rows0, sem0)
            g1 = pltpu.async_copy(
                table_hbm.at[idx_v.at[pl.ds(o1, chunk)]], rows1, sem1)
            g0.wait()
            pltpu.sync_copy(rows0, out_hbm.at[pl.ds(base + o0, chunk)])
            g1.wait()
            pltpu.sync_copy(rows1, out_hbm.at[pl.ds(base + o1, chunk)])
            return carry

        if n_chunks > 1:
            lax.fori_loop(0, n_chunks // 2, pair, 0)
        if n_chunks % 2:
            o = (n_chunks - 1) * chunk
            pltpu.async_copy(
                table_hbm.at[idx_v.at[pl.ds(o, chunk)]], rows0, sem0).wait()
            pltpu.sync_copy(rows0, out_hbm.at[pl.ds(base + o, chunk)])

    return gk(table, idx)


def _sa_tail_kernel(g_ref, q_ref, wq_ref, w2_ref, b2_ref, w3_ref, b3_ref,
                    o_ref, *, ns):
    qproj = jnp.dot(q_ref[...], wq_ref[...],
                    preferred_element_type=jnp.float32)
    bm, c1 = qproj.shape
    g = g_ref[...]
    x = g.reshape(bm, ns, c1) - qproj[:, None, :]
    x = jnp.maximum(x, 0.0).reshape(bm * ns, c1)
    h = jnp.maximum(jnp.dot(x, w2_ref[...],
                            preferred_element_type=jnp.float32) + b2_ref[...], 0.0)
    h = jnp.maximum(jnp.dot(h, w3_ref[...],
                            preferred_element_type=jnp.float32) + b3_ref[...], 0.0)
    o_ref[...] = jnp.max(h.reshape(bm, ns, h.shape[1]), axis=1)


def _sa_tail(g, new_pos, ws, bs, ns, bm):
    m = new_pos.shape[0]
    bm = min(bm, m)
    c1 = ws[0].shape[1]
    c3 = ws[2].shape[1]
    return pl.pallas_call(
        functools.partial(_sa_tail_kernel, ns=ns),
        grid=(m // bm,),
        in_specs=[pl.BlockSpec((bm * ns, c1), lambda i: (i, 0)),
                  pl.BlockSpec((bm, 3), lambda i: (i, 0)),
                  pl.BlockSpec((3, c1), lambda i: (0, 0)),
                  pl.BlockSpec(ws[1].shape, lambda i: (0, 0)),
                  pl.BlockSpec((1, ws[1].shape[1]), lambda i: (0, 0)),
                  pl.BlockSpec(ws[2].shape, lambda i: (0, 0)),
                  pl.BlockSpec((1, c3), lambda i: (0, 0))],
        out_specs=pl.BlockSpec((bm, c3), lambda i: (i, 0)),
        out_shape=jax.ShapeDtypeStruct((m, c3), jnp.float32),
    )(g, new_pos, ws[0][:3], ws[1], bs[1].reshape(1, -1),
      ws[2], bs[2].reshape(1, -1))


def _sa(pos, feat, idx, ws, bs, bm):
    m = pos.shape[0] // 4
    new_pos = pos[:m]
    w1, b1, w2 = ws[0], bs[0], ws[1]
    c1 = w1.shape[1]
    if c1 % 128:
        pad = 128 - c1 % 128
        w1 = jnp.pad(w1, ((0, 0), (0, pad)))
        b1 = jnp.pad(b1, ((0, pad),))
        w2 = jnp.pad(w2, ((0, pad), (0, 0)))
    h = _linear(jnp.concatenate([pos, feat], axis=1), w1, b1)
    g = _sc_gather(h, idx.reshape(-1))
    f = _sa_tail(g, new_pos, [w1, w2, ws[2]], [b1, bs[1], bs[2]], 32, bm)
    return new_pos, f


def _fp_tail_kernel(*refs, n_layers, has_f1):
    rows_ref, dv_ref = refs[0], refs[1]
    pos_arg = 2
    f1 = None
    if has_f1:
        f1 = refs[pos_arg][...]
        pos_arg += 1
    wb = refs[pos_arg:pos_arg + 2 * n_layers]
    o_ref = refs[-1]

    dv = dv_ref[...]
    recip = 1.0 / (jnp.sqrt(dv) + 1e-8)
    w = recip / jnp.sum(recip, axis=1, keepdims=True)
    rows = rows_ref[...]
    bm = dv.shape[0]
    c2 = rows.shape[1]
    interp = jnp.sum(rows.reshape(bm, 3, c2) * w[:, :, None], axis=1)
    x = interp if f1 is None else jnp.concatenate([f1, interp], axis=1)
    for i in range(n_layers):
        x = jnp.maximum(
            jnp.dot(x, wb[2 * i][...],
                    preferred_element_type=jnp.float32) + wb[2 * i + 1][...],
            0.0)
    o_ref[...] = x


def _fp(pos1, feat1, feat2, idx, dv, ws, bs, bm):
    m = pos1.shape[0]
    rows = _sc_gather(feat2, idx.reshape(-1))
    bm = min(bm, m)
    c2 = feat2.shape[1]
    n_layers = len(ws)
    has_f1 = feat1 is not None
    cout = ws[-1].shape[1]

    in_specs = [pl.BlockSpec((bm * 3, c2), lambda i: (i, 0)),
                pl.BlockSpec((bm, 3), lambda i: (i, 0))]
    args = [rows, dv]
    if has_f1:
        in_specs.append(pl.BlockSpec((bm, feat1.shape[1]), lambda i: (i, 0)))
        args.append(feat1)
    for w, b in zip(ws, bs):
        in_specs.append(pl.BlockSpec(w.shape, lambda i: (0, 0)))
        in_specs.append(pl.BlockSpec((1, w.shape[1]), lambda i: (0, 0)))
        args.append(w)
        args.append(b.reshape(1, -1))

    return pl.pallas_call(
        functools.partial(_fp_tail_kernel, n_layers=n_layers, has_f1=has_f1),
        grid=(m // bm,),
        in_specs=in_specs,
        out_specs=pl.BlockSpec((bm, cout), lambda i: (i, 0)),
        out_shape=jax.ShapeDtypeStruct((m, cout), jnp.float32),
    )(*args)


def kernel(points, batch_size, sa1_w, sa1_b, sa2_w, sa2_b, sa3_w, sa3_b,
           sa4_w, sa4_b, fp4_w, fp4_b, fp3_w, fp3_b, fp2_w, fp2_b,
           fp1_w, fp1_b):
    pos = points[:, 1:4]
    feat = points[:, 4:]
    p1, p2, p3, p4 = pos[:4096], pos[:1024], pos[:256], pos[:64]

    i_sa1, _ = _knn_filtered(p1, pos, 32, 128, 128)
    i_fp1, d_fp1 = _knn_filtered(pos, p1, 3, 256, 512)
    i_sa2, _ = _knn(p2, p1, 32, 128)
    i_sa3, _ = _knn(p3, p2, 32, 256)
    i_sa4, _ = _knn(p4, p3, 32, 64)
    i_fp2, d_fp2 = _knn(p1, p2, 3, 256)
    i_fp3, d_fp3 = _knn(p2, p3, 3, 256)
    i_fp4, d_fp4 = _knn(p3, p4, 3, 256)

    _, feat1 = _sa(pos, feat, i_sa1, sa1_w, sa1_b, bm=128)
    _, feat2 = _sa(p1, feat1, i_sa2, sa2_w, sa2_b, bm=64)
    _, feat3 = _sa(p2, feat2, i_sa3, sa3_w, sa3_b, bm=64)
    _, feat4 = _sa(p3, feat3, i_sa4, sa4_w, sa4_b, bm=16)
    feat3 = _fp(p3, feat3, feat4, i_fp4, d_fp4, fp4_w, fp4_b, bm=64)
    feat2 = _fp(p2, feat2, feat3, i_fp3, d_fp3, fp3_w, fp3_b, bm=128)
    feat1 = _fp(p1, feat1, feat2, i_fp2, d_fp2, fp2_w, fp2_b, bm=128)
    feat0 = _fp(pos, None, feat1, i_fp1, d_fp1, fp1_w, fp1_b, bm=256)
    return feat0
